# baseline (speedup 1.0000x reference)
import jax
import jax.numpy as jnp
from jax.experimental import pallas as pl
from jax.experimental.pallas import tpu as pltpu


def _w_chunk_index(p, t, n_t):
    return jnp.where(p == 0, t, jnp.where(p == 1, n_t + t, 2 * n_t - 1))


def _linear_kernel(x_ref, w_ref, b_ref, o_ref, w_bf16):
    p = pl.program_id(0)
    t = pl.program_id(1)
    n_t = pl.num_programs(1)
    wc = w_ref.shape[0]
    bn = o_ref.shape[1]

    @pl.when(p >= 1)
    def _matmul():
        half = jnp.where(p == 2, 1, 0)
        acc = jax.lax.dot_general(
            x_ref[...].astype(jnp.bfloat16),
            w_bf16[pl.ds(half * bn, bn), :],
            dimension_numbers=(((1,), (1,)), ((), ())),
            preferred_element_type=jnp.float32,
        )
        o_ref[...] = acc + b_ref[...]

    @pl.when(p <= 1)
    def _fill_w():
        c = _w_chunk_index(p, t, n_t)
        w_bf16[pl.ds(c * wc, wc), :] = w_ref[...].astype(jnp.bfloat16)


def kernel(x, weight, bias):
    B, K = x.shape
    N = weight.shape[0]

    b2 = bias.astype(jnp.float32).reshape(1, N)

    bm = 512
    bn = N // 2
    n_t = B // bm
    wc = N // (2 * n_t)

    out = pl.pallas_call(
        _linear_kernel,
        out_shape=jax.ShapeDtypeStruct((B, N), jnp.float32),
        grid=(3, n_t),
        in_specs=[
            pl.BlockSpec((bm, K), lambda p, t: (jnp.where(p == 0, 0, t), 0)),
            pl.BlockSpec(
                (wc, K),
                lambda p, t: (_w_chunk_index(p, t, pl.num_programs(1)), 0),
            ),
            pl.BlockSpec((1, bn), lambda p, t: (0, jnp.where(p == 2, 1, 0))),
        ],
        out_specs=pl.BlockSpec(
            (bm, bn),
            lambda p, t: (jnp.where(p == 0, 0, t), jnp.where(p == 2, 1, 0)),
        ),
        scratch_shapes=[pltpu.VMEM((N, K), jnp.bfloat16)],
        compiler_params=pltpu.CompilerParams(
            dimension_semantics=("arbitrary", "arbitrary"),
            vmem_limit_bytes=64 * 1024 * 1024,
        ),
        cost_estimate=pl.CostEstimate(
            flops=2 * B * N * K,
            transcendentals=0,
            bytes_accessed=4 * (2 * B * K + N * K + B * N + N),
        ),
    )(x, weight, b2)
    return out


# speedup vs baseline: 1.0015x; 1.0015x over previous
import jax
import jax.numpy as jnp
from jax.experimental import pallas as pl
from jax.experimental.pallas import tpu as pltpu


def _w_chunk_index(p, t, n_t):
    return jnp.where(p == 0, t, jnp.where(p == 1, n_t + t, 2 * n_t - 1))


def _linear_kernel(x_ref, w_ref, b_ref, o_ref, w_bf16):
    p = pl.program_id(0)
    t = pl.program_id(1)
    n_t = pl.num_programs(1)
    wc = w_ref.shape[0]
    bn = o_ref.shape[1]

    @pl.when(p <= 1)
    def _fill_w():
        c = _w_chunk_index(p, t, n_t)
        w_bf16[pl.ds(c * wc, wc), :] = w_ref[...].astype(jnp.bfloat16)

    @pl.when(p >= 1)
    def _matmul():
        half = jnp.where(p == 2, 1, 0)
        acc = jax.lax.dot_general(
            x_ref[...].astype(jnp.bfloat16),
            w_bf16[pl.ds(half * bn, bn), :],
            dimension_numbers=(((1,), (1,)), ((), ())),
            preferred_element_type=jnp.float32,
        )
        o_ref[...] = acc + b_ref[...]


def kernel(x, weight, bias):
    B, K = x.shape
    N = weight.shape[0]

    b2 = bias.astype(jnp.float32).reshape(1, N)

    bm = 512
    bn = N // 2
    n_t = B // bm
    wc = N // (2 * n_t)

    out = pl.pallas_call(
        _linear_kernel,
        out_shape=jax.ShapeDtypeStruct((B, N), jnp.float32),
        grid=(3, n_t),
        in_specs=[
            pl.BlockSpec((bm, K), lambda p, t: (jnp.where(p == 0, 0, t), 0)),
            pl.BlockSpec(
                (wc, K),
                lambda p, t: (_w_chunk_index(p, t, pl.num_programs(1)), 0),
            ),
            pl.BlockSpec((1, bn), lambda p, t: (0, jnp.where(p == 2, 1, 0))),
        ],
        out_specs=pl.BlockSpec(
            (bm, bn),
            lambda p, t: (jnp.where(p == 0, 0, t), jnp.where(p == 2, 1, 0)),
        ),
        scratch_shapes=[pltpu.VMEM((N, K), jnp.bfloat16)],
        compiler_params=pltpu.CompilerParams(
            dimension_semantics=("arbitrary", "arbitrary"),
            vmem_limit_bytes=64 * 1024 * 1024,
        ),
        cost_estimate=pl.CostEstimate(
            flops=2 * B * N * K,
            transcendentals=0,
            bytes_accessed=4 * (2 * B * K + N * K + B * N + N),
        ),
    )(x, weight, b2)
    return out
